# baseline (speedup 1.0000x reference)
import functools

import jax
import jax.numpy as jnp
from jax import lax
from jax.experimental import pallas as pl
from jax.experimental.pallas import tpu as pltpu
from jax.experimental.pallas import tpu_sc as plsc

N = 10000
NPAD = 10240
F_IN = 128
F_HID = 128
F_OUT = 64
NC = 2
NS = 16
NW = NC * NS
EPC = 128

ROWS_PER_TILE = NPAD // NS


def _mesh():
    return plsc.VectorSubcoreMesh(core_axis_name="c", subcore_axis_name="s")


def _sc_prop(g, src, dst, zrows, col_split=False, k2=4, dup_table=False):
    f = g.shape[-1]
    nch = src.shape[1]

    @functools.partial(
        pl.kernel,
        out_type=jax.ShapeDtypeStruct((NC, NPAD, f), jnp.float32),
        mesh=_mesh(),
        compiler_params=pltpu.CompilerParams(use_tc_tiling_on_sc=False),
        scratch_types=[
            pltpu.VMEM_SHARED((NPAD, f), jnp.float32),
            pltpu.VMEM((2, k2 * EPC), jnp.int32),
            pltpu.VMEM((nch, k2 * EPC), jnp.int32),
            pltpu.VMEM((2, k2 * EPC, f), jnp.float32),
            [pltpu.SemaphoreType.DMA] * 2,
            [pltpu.SemaphoreType.DMA] * 2,
            [pltpu.SemaphoreType.DMA] * 2,
        ],
    )
    def k(g_hbm, src_hbm, dst_hbm, z_hbm, out_hbm, acc, isrc, idst, rows,
          gsems, ssems, isems):
        c = lax.axis_index("c")
        s = lax.axis_index("s")
        wid = s if col_split else s * NC + (1 - c)
        tab = g_hbm.at[c] if (col_split or dup_table) else g_hbm
        r0 = s * ROWS_PER_TILE
        pltpu.sync_copy(z_hbm.at[pl.ds(r0, ROWS_PER_TILE)],
                        acc.at[pl.ds(r0, ROWS_PER_TILE)])
        pltpu.sync_copy(dst_hbm.at[wid], idst)
        pltpu.sync_copy(src_hbm.at[wid, 0], isrc.at[0])
        plsc.subcore_barrier()

        g0 = pltpu.async_copy(tab.at[isrc.at[0]], rows.at[0], gsems[0])
        i1 = pltpu.async_copy(src_hbm.at[wid, 1], isrc.at[1], isems[1])

        def group(tg, _):
            for b in range(2):
                t = tg * 2 + b
                nb = 1 - b

                @pl.when(t >= 1)
                def _():
                    pltpu.make_async_copy(rows.at[nb], acc.at[idst.at[t - 1]],
                                          ssems[nb]).wait()

                @pl.when(t + 1 < nch)
                def _():
                    pltpu.make_async_copy(src_hbm.at[wid, t + 1], isrc.at[nb],
                                          isems[nb]).wait()
                    pltpu.async_copy(tab.at[isrc.at[nb]], rows.at[nb],
                                     gsems[nb])

                pltpu.make_async_copy(tab.at[isrc.at[b]], rows.at[b],
                                      gsems[b]).wait()
                pltpu.async_copy(rows.at[b], acc.at[idst.at[t]], ssems[b],
                                 add=True)

                @pl.when(t + 2 < nch)
                def _():
                    pltpu.async_copy(src_hbm.at[wid, t + 2], isrc.at[b],
                                     isems[b])
            return 0

        lax.fori_loop(0, nch // 2, group, 0)
        pltpu.make_async_copy(rows.at[1], acc.at[idst.at[nch - 1]],
                              ssems[1]).wait()
        plsc.subcore_barrier()
        pltpu.sync_copy(acc.at[pl.ds(r0, ROWS_PER_TILE)],
                        out_hbm.at[c, pl.ds(r0, ROWS_PER_TILE)])

    return k(g, src, dst, zrows)


_BM = 512


def _tc_stage1(xp, w1, ppd):
    def body(x_ref, w_ref, p_ref, dis_ref, h_ref, g_ref):
        deg = p_ref[0, :, 0] + p_ref[1, :, 0] + 1.0
        disv = lax.rsqrt(deg)
        h = jnp.dot(x_ref[...], w_ref[...], preferred_element_type=jnp.float32)
        dis_ref[...] = disv[None, :]
        h_ref[...] = h
        gv = disv[:, None] * h
        hf = F_HID // 2
        g_ref[0] = gv[:, :hf]
        g_ref[1] = gv[:, hf:]

    grid = (NPAD // _BM,)
    return pl.pallas_call(
        body,
        grid=grid,
        in_specs=[
            pl.BlockSpec((_BM, F_IN), lambda i: (i, 0)),
            pl.BlockSpec((F_IN, F_HID), lambda i: (0, 0)),
            pl.BlockSpec((NC, _BM, 16), lambda i: (0, i, 0)),
        ],
        out_specs=[
            pl.BlockSpec((1, _BM), lambda i: (0, i)),
            pl.BlockSpec((_BM, F_HID), lambda i: (i, 0)),
            pl.BlockSpec((NC, _BM, F_HID // 2), lambda i: (0, i, 0)),
        ],
        out_shape=[
            jax.ShapeDtypeStruct((1, NPAD), jnp.float32),
            jax.ShapeDtypeStruct((NPAD, F_HID), jnp.float32),
            jax.ShapeDtypeStruct((NC, NPAD, F_HID // 2), jnp.float32),
        ],
    )(xp, w1, ppd)


def _tc_stage2(pp, dis, h1, w2, b1):
    def body(p_ref, d_ref, h_ref, w_ref, b_ref, h2_ref, g2_ref):
        disv = d_ref[0, :]
        p = jnp.concatenate([p_ref[0], p_ref[1]], axis=-1)
        x1 = disv[:, None] * p + (disv * disv)[:, None] * h_ref[...] + b_ref[...]
        x1 = jnp.maximum(x1, 0.0)
        h2 = jnp.dot(x1, w_ref[...], preferred_element_type=jnp.float32)
        h2_ref[...] = h2
        g2 = disv[:, None] * h2
        g2_ref[0] = g2
        g2_ref[1] = g2

    grid = (NPAD // _BM,)
    return pl.pallas_call(
        body,
        grid=grid,
        in_specs=[
            pl.BlockSpec((NC, _BM, F_HID // 2), lambda i: (0, i, 0)),
            pl.BlockSpec((1, _BM), lambda i: (0, i)),
            pl.BlockSpec((_BM, F_HID), lambda i: (i, 0)),
            pl.BlockSpec((F_HID, F_OUT), lambda i: (0, 0)),
            pl.BlockSpec((1, F_HID), lambda i: (0, 0)),
        ],
        out_specs=[
            pl.BlockSpec((_BM, F_OUT), lambda i: (i, 0)),
            pl.BlockSpec((NC, _BM, F_OUT), lambda i: (0, i, 0)),
        ],
        out_shape=[
            jax.ShapeDtypeStruct((NPAD, F_OUT), jnp.float32),
            jax.ShapeDtypeStruct((NC, NPAD, F_OUT), jnp.float32),
        ],
    )(pp, dis, h1, w2, b1)


def _tc_stage3(pp, dis, h2, b2):
    def body(p_ref, d_ref, h_ref, b_ref, o_ref):
        disv = d_ref[0, :]
        p = p_ref[0] + p_ref[1]
        o_ref[...] = (disv[:, None] * p
                      + (disv * disv)[:, None] * h_ref[...] + b_ref[...])

    grid = (NPAD // _BM,)
    return pl.pallas_call(
        body,
        grid=grid,
        in_specs=[
            pl.BlockSpec((NC, _BM, F_OUT), lambda i: (0, i, 0)),
            pl.BlockSpec((1, _BM), lambda i: (0, i)),
            pl.BlockSpec((_BM, F_OUT), lambda i: (i, 0)),
            pl.BlockSpec((1, F_OUT), lambda i: (0, 0)),
        ],
        out_specs=pl.BlockSpec((_BM, F_OUT), lambda i: (i, 0)),
        out_shape=jax.ShapeDtypeStruct((NPAD, F_OUT), jnp.float32),
    )(pp, dis, h2, b2)


K2_DEG = 10
K2_PROP = 4


def _pad_edges(ei, epad):
    e = ei.shape[1]
    fill = jnp.full((epad - e,), N, dtype=ei.dtype)
    return (jnp.concatenate([ei[0], fill]), jnp.concatenate([ei[1], fill]))


def _rs(a, nblocks, k2):
    return a.reshape(nblocks, -1, k2 * EPC)


def _branch(xp, fsrc, fdst, ppd, w1, b1, w2, b2, z64):
    dis, h1, g1 = _tc_stage1(xp, w1, ppd)
    pp1 = _sc_prop(g1, _rs(fsrc, NS, K2_PROP), _rs(fdst, NS, K2_PROP), z64,
                   col_split=True, k2=K2_PROP)
    h2, g2 = _tc_stage2(pp1, dis, h1, w2, b1.reshape(1, F_HID))
    pp2 = _sc_prop(g2, _rs(fsrc, NW, K2_PROP), _rs(fdst, NW, K2_PROP), z64,
                   k2=K2_PROP, dup_table=True)
    return _tc_stage3(pp2, dis, h2, b2.reshape(1, F_OUT))


def kernel(x_data_matrix, x_edge_index, y_data_matrix, y_edge_index,
           W1x, b1x, W2x, b2x, W1y, b1y, W2y, b2y):
    e = x_edge_index.shape[1]
    chunk = NW * EPC * K2_DEG
    epad = ((e + chunk - 1) // chunk) * chunk

    fsrc_x, fdst_x = _pad_edges(x_edge_index, epad)
    fsrc_y, fdst_y = _pad_edges(y_edge_index, epad)

    xp = jnp.pad(x_data_matrix, ((0, NPAD - N), (0, 0)))
    yp = jnp.pad(y_data_matrix, ((0, NPAD - N), (0, 0)))

    z64 = jnp.zeros((NPAD, F_OUT), jnp.float32)
    z16 = jnp.zeros((NPAD, 16), jnp.float32)
    ones16 = jnp.ones((NC, NPAD, 16), jnp.float32)

    ppd_x = _sc_prop(ones16, _rs(fsrc_x, NW, K2_DEG), _rs(fdst_x, NW, K2_DEG),
                     z16, k2=K2_DEG, dup_table=True)
    ppd_y = _sc_prop(ones16, _rs(fsrc_y, NW, K2_DEG), _rs(fdst_y, NW, K2_DEG),
                     z16, k2=K2_DEG, dup_table=True)

    out_x = _branch(xp, fsrc_x, fdst_x, ppd_x, W1x, b1x, W2x, b2x, z64)
    out_y = _branch(yp, fsrc_y, fdst_y, ppd_y, W1y, b1y, W2y, b2y, z64)
    return (out_x[:N], out_y[:N])


# speedup vs baseline: 2.8955x; 2.8955x over previous
import functools

import jax
import jax.numpy as jnp
from jax import lax
from jax.experimental import pallas as pl
from jax.experimental.pallas import tpu as pltpu
from jax.experimental.pallas import tpu_sc as plsc

N = 10000
NPAD = 10240
F_IN = 128
F_HID = 128
F_OUT = 64
NC = 2
NS = 16
NW = NC * NS
EPC = 128

ROWS_PER_TILE = NPAD // NS


def _mesh():
    return plsc.VectorSubcoreMesh(core_axis_name="c", subcore_axis_name="s")


def _sc_prop(g, src, dst, zrows, col_split=False, k2=4, dup_table=False):
    f = g.shape[-1]
    nch = src.shape[1]

    @functools.partial(
        pl.kernel,
        out_type=jax.ShapeDtypeStruct((NC, NPAD, f), jnp.float32),
        mesh=_mesh(),
        compiler_params=pltpu.CompilerParams(use_tc_tiling_on_sc=False),
        scratch_types=[
            pltpu.VMEM_SHARED((NPAD, f), jnp.float32),
            pltpu.VMEM((2, k2 * EPC), jnp.int32),
            pltpu.VMEM((nch, k2 * EPC), jnp.int32),
            pltpu.VMEM((2, k2 * EPC, f), jnp.float32),
            [pltpu.SemaphoreType.DMA] * 2,
            [pltpu.SemaphoreType.DMA] * 2,
            [pltpu.SemaphoreType.DMA] * 2,
        ],
    )
    def k(g_hbm, src_hbm, dst_hbm, z_hbm, out_hbm, acc, isrc, idst, rows,
          gsems, ssems, isems):
        c = lax.axis_index("c")
        s = lax.axis_index("s")
        wid = s if col_split else s * NC + c
        tab = g_hbm.at[c] if (col_split or dup_table) else g_hbm
        r0 = s * ROWS_PER_TILE
        pltpu.sync_copy(z_hbm.at[pl.ds(r0, ROWS_PER_TILE)],
                        acc.at[pl.ds(r0, ROWS_PER_TILE)])
        pltpu.sync_copy(dst_hbm.at[wid], idst)
        pltpu.sync_copy(src_hbm.at[wid, 0], isrc.at[0])
        plsc.subcore_barrier()

        g0 = pltpu.async_copy(tab.at[isrc.at[0]], rows.at[0], gsems[0])
        i1 = pltpu.async_copy(src_hbm.at[wid, 1], isrc.at[1], isems[1])

        def group(tg, _):
            for b in range(2):
                t = tg * 2 + b
                nb = 1 - b

                @pl.when(t >= 1)
                def _():
                    pltpu.make_async_copy(rows.at[nb], acc.at[idst.at[t - 1]],
                                          ssems[nb]).wait()

                @pl.when(t + 1 < nch)
                def _():
                    pltpu.make_async_copy(src_hbm.at[wid, t + 1], isrc.at[nb],
                                          isems[nb]).wait()
                    pltpu.async_copy(tab.at[isrc.at[nb]], rows.at[nb],
                                     gsems[nb])

                pltpu.make_async_copy(tab.at[isrc.at[b]], rows.at[b],
                                      gsems[b]).wait()
                pltpu.async_copy(rows.at[b], acc.at[idst.at[t]], ssems[b],
                                 add=True)

                @pl.when(t + 2 < nch)
                def _():
                    pltpu.async_copy(src_hbm.at[wid, t + 2], isrc.at[b],
                                     isems[b])
            return 0

        lax.fori_loop(0, nch // 2, group, 0)
        pltpu.make_async_copy(rows.at[1], acc.at[idst.at[nch - 1]],
                              ssems[1]).wait()
        plsc.subcore_barrier()
        pltpu.sync_copy(acc.at[pl.ds(r0, ROWS_PER_TILE)],
                        out_hbm.at[c, pl.ds(r0, ROWS_PER_TILE)])

    return k(g, src, dst, zrows)


_BM = 512


def _tc_stage1(xp, w1, ppd):
    def body(x_ref, w_ref, p_ref, dis_ref, h_ref, g_ref):
        deg = p_ref[0, :, 0] + p_ref[1, :, 0] + 1.0
        disv = lax.rsqrt(deg)
        h = jnp.dot(x_ref[...], w_ref[...], preferred_element_type=jnp.float32)
        dis_ref[...] = disv[None, :]
        h_ref[...] = h
        gv = disv[:, None] * h
        hf = F_HID // 2
        g_ref[0] = gv[:, :hf]
        g_ref[1] = gv[:, hf:]

    grid = (NPAD // _BM,)
    return pl.pallas_call(
        body,
        grid=grid,
        in_specs=[
            pl.BlockSpec((_BM, F_IN), lambda i: (i, 0)),
            pl.BlockSpec((F_IN, F_HID), lambda i: (0, 0)),
            pl.BlockSpec((NC, _BM, 16), lambda i: (0, i, 0)),
        ],
        out_specs=[
            pl.BlockSpec((1, _BM), lambda i: (0, i)),
            pl.BlockSpec((_BM, F_HID), lambda i: (i, 0)),
            pl.BlockSpec((NC, _BM, F_HID // 2), lambda i: (0, i, 0)),
        ],
        out_shape=[
            jax.ShapeDtypeStruct((1, NPAD), jnp.float32),
            jax.ShapeDtypeStruct((NPAD, F_HID), jnp.float32),
            jax.ShapeDtypeStruct((NC, NPAD, F_HID // 2), jnp.float32),
        ],
    )(xp, w1, ppd)


def _tc_stage2(pp, dis, h1, w2, b1):
    def body(p_ref, d_ref, h_ref, w_ref, b_ref, h2_ref, g2_ref):
        disv = d_ref[0, :]
        p = jnp.concatenate([p_ref[0], p_ref[1]], axis=-1)
        x1 = disv[:, None] * p + (disv * disv)[:, None] * h_ref[...] + b_ref[...]
        x1 = jnp.maximum(x1, 0.0)
        h2 = jnp.dot(x1, w_ref[...], preferred_element_type=jnp.float32)
        h2_ref[...] = h2
        g2 = disv[:, None] * h2
        g2_ref[0] = g2
        g2_ref[1] = g2

    grid = (NPAD // _BM,)
    return pl.pallas_call(
        body,
        grid=grid,
        in_specs=[
            pl.BlockSpec((NC, _BM, F_HID // 2), lambda i: (0, i, 0)),
            pl.BlockSpec((1, _BM), lambda i: (0, i)),
            pl.BlockSpec((_BM, F_HID), lambda i: (i, 0)),
            pl.BlockSpec((F_HID, F_OUT), lambda i: (0, 0)),
            pl.BlockSpec((1, F_HID), lambda i: (0, 0)),
        ],
        out_specs=[
            pl.BlockSpec((_BM, F_OUT), lambda i: (i, 0)),
            pl.BlockSpec((NC, _BM, F_OUT), lambda i: (0, i, 0)),
        ],
        out_shape=[
            jax.ShapeDtypeStruct((NPAD, F_OUT), jnp.float32),
            jax.ShapeDtypeStruct((NC, NPAD, F_OUT), jnp.float32),
        ],
    )(pp, dis, h1, w2, b1)


def _tc_stage3(pp, dis, h2, b2):
    def body(p_ref, d_ref, h_ref, b_ref, o_ref):
        disv = d_ref[0, :]
        p = p_ref[0] + p_ref[1]
        o_ref[...] = (disv[:, None] * p
                      + (disv * disv)[:, None] * h_ref[...] + b_ref[...])

    grid = (NPAD // _BM,)
    return pl.pallas_call(
        body,
        grid=grid,
        in_specs=[
            pl.BlockSpec((NC, _BM, F_OUT), lambda i: (0, i, 0)),
            pl.BlockSpec((1, _BM), lambda i: (0, i)),
            pl.BlockSpec((_BM, F_OUT), lambda i: (i, 0)),
            pl.BlockSpec((1, F_OUT), lambda i: (0, 0)),
        ],
        out_specs=pl.BlockSpec((_BM, F_OUT), lambda i: (i, 0)),
        out_shape=jax.ShapeDtypeStruct((NPAD, F_OUT), jnp.float32),
    )(pp, dis, h2, b2)


K2_DEG = 10
K2_PROP = 4


def _pad_edges(ei, epad):
    e = ei.shape[1]
    fill = (jnp.arange(epad - e, dtype=ei.dtype) % (NPAD - N)) + N
    return (jnp.concatenate([ei[0], fill]), jnp.concatenate([ei[1], fill]))


def _rs(a, nblocks, k2):
    return a.reshape(nblocks, -1, k2 * EPC)


def _branch(xp, fsrc, fdst, ppd, w1, b1, w2, b2, z64):
    dis, h1, g1 = _tc_stage1(xp, w1, ppd)
    pp1 = _sc_prop(g1, _rs(fsrc, NS, K2_PROP), _rs(fdst, NS, K2_PROP), z64,
                   col_split=True, k2=K2_PROP)
    h2, g2 = _tc_stage2(pp1, dis, h1, w2, b1.reshape(1, F_HID))
    pp2 = _sc_prop(g2, _rs(fsrc, NW, K2_PROP), _rs(fdst, NW, K2_PROP), z64,
                   k2=K2_PROP, dup_table=True)
    return _tc_stage3(pp2, dis, h2, b2.reshape(1, F_OUT))


def kernel(x_data_matrix, x_edge_index, y_data_matrix, y_edge_index,
           W1x, b1x, W2x, b2x, W1y, b1y, W2y, b2y):
    e = x_edge_index.shape[1]
    chunk = NW * EPC * K2_DEG
    epad = ((e + chunk - 1) // chunk) * chunk

    fsrc_x, fdst_x = _pad_edges(x_edge_index, epad)
    fsrc_y, fdst_y = _pad_edges(y_edge_index, epad)

    xp = jnp.pad(x_data_matrix, ((0, NPAD - N), (0, 0)))
    yp = jnp.pad(y_data_matrix, ((0, NPAD - N), (0, 0)))

    z64 = jnp.zeros((NPAD, F_OUT), jnp.float32)
    z16 = jnp.zeros((NPAD, 16), jnp.float32)
    ones16 = jnp.ones((NC, NPAD, 16), jnp.float32)

    ppd_x = _sc_prop(ones16, _rs(fsrc_x, NW, K2_DEG), _rs(fdst_x, NW, K2_DEG),
                     z16, k2=K2_DEG, dup_table=True)
    ppd_y = _sc_prop(ones16, _rs(fsrc_y, NW, K2_DEG), _rs(fdst_y, NW, K2_DEG),
                     z16, k2=K2_DEG, dup_table=True)

    out_x = _branch(xp, fsrc_x, fdst_x, ppd_x, W1x, b1x, W2x, b2x, z64)
    out_y = _branch(yp, fsrc_y, fdst_y, ppd_y, W1y, b1y, W2y, b2y, z64)
    return (out_x[:N], out_y[:N])
